# baseline (speedup 1.0000x reference)
import functools

import jax
import jax.numpy as jnp
from jax.experimental import pallas as pl
from jax.experimental.pallas import tpu as pltpu

TS = 1024
HS = TS // 2
NSLOT = 2


def _adapter_body(idx_ref, xl_ref, xh_ref, dw_hbm, db_hbm, uw_hbm,
                  o_hbm, o_buf, dwv, dbv, uwv, sem_o, sem_w,
                  *, B, S, C, D, SB, T):
    mi = pl.program_id(0)
    b = pl.program_id(1)
    s = pl.program_id(2)
    t = (mi * pl.num_programs(1) + b) * SB + s
    slot = t % NSLOT
    CH = C // 2

    @pl.when(t == 0)
    def _gather_weights():
        for bb in range(B):
            e = idx_ref[bb]
            pltpu.make_async_copy(dw_hbm.at[0, e], dwv.at[bb], sem_w).start()
            pltpu.make_async_copy(db_hbm.at[0, e], dbv.at[bb], sem_w).start()
            pltpu.make_async_copy(uw_hbm.at[0, e], uwv.at[bb], sem_w).start()
        for bb in range(B):
            e = idx_ref[bb]
            pltpu.make_async_copy(dw_hbm.at[0, e], dwv.at[bb], sem_w).wait()
            pltpu.make_async_copy(db_hbm.at[0, e], dbv.at[bb], sem_w).wait()
            pltpu.make_async_copy(uw_hbm.at[0, e], uwv.at[bb], sem_w).wait()

    xl = xl_ref[0]
    xh = xh_ref[0]
    dwl = dwv[b, :CH]
    dwh = dwv[b, CH:]
    db = dbv[b]
    uw = uwv[b]

    z = (
        jnp.dot(xl, dwl, preferred_element_type=jnp.float32)
        + jnp.dot(xh, dwh, preferred_element_type=jnp.float32)
        + db[None, :]
    )
    z = z * jax.nn.sigmoid(z)

    @pl.when(t >= NSLOT)
    def _wait_slot():
        tp = t - NSLOT
        bp = tp // SB
        sp = tp % SB
        base = sp * TS
        pltpu.make_async_copy(
            o_buf.at[slot, pl.ds(0, HS), :],
            o_hbm.at[0, bp, pl.ds(base, HS), :],
            sem_o.at[slot, 0],
        ).wait()
        pltpu.make_async_copy(
            o_buf.at[slot, pl.ds(HS, HS), :],
            o_hbm.at[0, bp, pl.ds(base + HS, HS), :],
            sem_o.at[slot, 1],
        ).wait()

    o_buf[slot, pl.ds(0, HS), :] = jnp.dot(
        z[:HS], uw, preferred_element_type=jnp.float32
    )
    pltpu.make_async_copy(
        o_buf.at[slot, pl.ds(0, HS), :],
        o_hbm.at[0, b, pl.ds(s * TS, HS), :],
        sem_o.at[slot, 0],
    ).start()

    o_buf[slot, pl.ds(HS, HS), :] = jnp.dot(
        z[HS:], uw, preferred_element_type=jnp.float32
    )
    pltpu.make_async_copy(
        o_buf.at[slot, pl.ds(HS, HS), :],
        o_hbm.at[0, b, pl.ds(s * TS + HS, HS), :],
        sem_o.at[slot, 1],
    ).start()

    @pl.when(t == T - 1)
    def _drain():
        for tq in range(max(0, T - NSLOT), T):
            bq, sq = tq // SB, tq % SB
            for h in range(2):
                pltpu.make_async_copy(
                    o_buf.at[tq % NSLOT, pl.ds(h * HS, HS), :],
                    o_hbm.at[0, bq, pl.ds(sq * TS + h * HS, HS), :],
                    sem_o.at[tq % NSLOT, h],
                ).wait()


@jax.jit
def kernel(x, expert_index, down_w, down_b, up_w):
    B, S, C = x.shape
    M, N, _, D = down_w.shape
    CH = C // 2
    SB = S // TS
    T = M * B * SB

    idx = expert_index.astype(jnp.int32).reshape(M * B)

    grid_spec = pltpu.PrefetchScalarGridSpec(
        num_scalar_prefetch=1,
        grid=(M, B, SB),
        in_specs=[
            pl.BlockSpec((1, TS, CH), lambda mm, b, s, i: (b, s, 0)),
            pl.BlockSpec((1, TS, CH), lambda mm, b, s, i: (b, s, 1)),
            pl.BlockSpec(memory_space=pltpu.MemorySpace.HBM),
            pl.BlockSpec(memory_space=pltpu.MemorySpace.HBM),
            pl.BlockSpec(memory_space=pltpu.MemorySpace.HBM),
        ],
        out_specs=pl.BlockSpec(memory_space=pltpu.MemorySpace.HBM),
        scratch_shapes=[
            pltpu.VMEM((NSLOT, TS, C), jnp.float32),
            pltpu.VMEM((B, C, D), jnp.float32),
            pltpu.VMEM((B, D), jnp.float32),
            pltpu.VMEM((B, D, C), jnp.float32),
            pltpu.SemaphoreType.DMA((NSLOT, 2)),
            pltpu.SemaphoreType.DMA,
        ],
    )

    out = pl.pallas_call(
        functools.partial(_adapter_body, B=B, S=S, C=C, D=D, SB=SB, T=T),
        grid_spec=grid_spec,
        out_shape=jax.ShapeDtypeStruct((M, B, S, C), jnp.float32),
        compiler_params=pltpu.CompilerParams(
            dimension_semantics=("arbitrary", "arbitrary", "arbitrary"),
        ),
    )(idx, x, x, down_w, down_b, up_w)
    return out


# speedup vs baseline: 1.0122x; 1.0122x over previous
import functools

import jax
import jax.numpy as jnp
from jax.experimental import pallas as pl
from jax.experimental.pallas import tpu as pltpu

TS = 1024
HS = TS // 2
NSLOT = 2


def _adapter_body(idx_ref, xl_ref, xh_ref, dw_hbm, db_hbm, uw_hbm,
                  o_hbm, o_buf, dwv, dbv, uwv, sem_o, sem_w,
                  *, B, S, C, D, SB, T):
    mi = pl.program_id(0)
    b = pl.program_id(1)
    s = pl.program_id(2)
    t = (mi * pl.num_programs(1) + b) * SB + s
    slot = t % NSLOT
    CH = C // 2

    @pl.when(t == 0)
    def _gather_weights():
        for bb in range(B):
            e = idx_ref[bb]
            pltpu.make_async_copy(dw_hbm.at[0, e], dwv.at[bb], sem_w).start()
            pltpu.make_async_copy(db_hbm.at[0, e], dbv.at[bb], sem_w).start()
            pltpu.make_async_copy(uw_hbm.at[0, e], uwv.at[bb], sem_w).start()
        for bb in range(B):
            e = idx_ref[bb]
            pltpu.make_async_copy(dw_hbm.at[0, e], dwv.at[bb], sem_w).wait()
            pltpu.make_async_copy(db_hbm.at[0, e], dbv.at[bb], sem_w).wait()
            pltpu.make_async_copy(uw_hbm.at[0, e], uwv.at[bb], sem_w).wait()

    xl = xl_ref[0]
    xh = xh_ref[0]
    dwl = dwv[b, :CH]
    dwh = dwv[b, CH:]
    db = dbv[b]
    uw = uwv[b]

    z = (
        jnp.dot(xl, dwl, preferred_element_type=jnp.float32)
        + jnp.dot(xh, dwh, preferred_element_type=jnp.float32)
        + db[None, :]
    )
    z = z * jax.nn.sigmoid(z)

    @pl.when(t >= NSLOT)
    def _wait_slot():
        tp = t - NSLOT
        bp = tp // SB
        sp = tp % SB
        base = sp * TS
        pltpu.make_async_copy(
            o_buf.at[slot, pl.ds(0, HS), :],
            o_hbm.at[0, bp, pl.ds(base, HS), :],
            sem_o.at[slot, 0],
        ).wait()
        pltpu.make_async_copy(
            o_buf.at[slot, pl.ds(HS, HS), :],
            o_hbm.at[0, bp, pl.ds(base + HS, HS), :],
            sem_o.at[slot, 1],
        ).wait()

    o_buf[slot, pl.ds(0, HS), :] = jnp.dot(
        z[:HS], uw, preferred_element_type=jnp.float32
    )
    pltpu.make_async_copy(
        o_buf.at[slot, pl.ds(0, HS), :],
        o_hbm.at[0, b, pl.ds(s * TS, HS), :],
        sem_o.at[slot, 0],
    ).start()

    o_buf[slot, pl.ds(HS, HS), :] = jnp.dot(
        z[HS:], uw, preferred_element_type=jnp.float32
    )
    pltpu.make_async_copy(
        o_buf.at[slot, pl.ds(HS, HS), :],
        o_hbm.at[0, b, pl.ds(s * TS + HS, HS), :],
        sem_o.at[slot, 1],
    ).start()

    @pl.when(t == T - 1)
    def _drain():
        for tq in range(max(0, T - NSLOT), T):
            bq, sq = tq // SB, tq % SB
            for h in range(2):
                pltpu.make_async_copy(
                    o_buf.at[tq % NSLOT, pl.ds(h * HS, HS), :],
                    o_hbm.at[0, bq, pl.ds(sq * TS + h * HS, HS), :],
                    sem_o.at[tq % NSLOT, h],
                ).wait()


@jax.jit
def kernel(x, expert_index, down_w, down_b, up_w):
    B, S, C = x.shape
    M, N, _, D = down_w.shape
    CH = C // 2
    SB = S // TS
    T = M * B * SB

    idx = expert_index.astype(jnp.int32).reshape(M * B)

    out = pl.pallas_call(
        functools.partial(_adapter_body, B=B, S=S, C=C, D=D, SB=SB, T=T),
        grid=(M, B, SB),
        in_specs=[
            pl.BlockSpec(memory_space=pltpu.MemorySpace.SMEM),
            pl.BlockSpec((1, TS, CH), lambda mm, b, s: (b, s, 0)),
            pl.BlockSpec((1, TS, CH), lambda mm, b, s: (b, s, 1)),
            pl.BlockSpec(memory_space=pltpu.MemorySpace.HBM),
            pl.BlockSpec(memory_space=pltpu.MemorySpace.HBM),
            pl.BlockSpec(memory_space=pltpu.MemorySpace.HBM),
        ],
        out_specs=pl.BlockSpec(memory_space=pltpu.MemorySpace.HBM),
        scratch_shapes=[
            pltpu.VMEM((NSLOT, TS, C), jnp.float32),
            pltpu.VMEM((B, C, D), jnp.float32),
            pltpu.VMEM((B, D), jnp.float32),
            pltpu.VMEM((B, D, C), jnp.float32),
            pltpu.SemaphoreType.DMA((NSLOT, 2)),
            pltpu.SemaphoreType.DMA,
        ],
        out_shape=jax.ShapeDtypeStruct((M, B, S, C), jnp.float32),
        compiler_params=pltpu.CompilerParams(
            dimension_semantics=("arbitrary", "arbitrary", "arbitrary"),
        ),
    )(idx, x, x, down_w, down_b, up_w)
    return out
